# baseline (speedup 1.0000x reference)
import functools
import numpy as np
import jax
import jax.numpy as jnp
from jax import lax
from jax.experimental import pallas as pl
from jax.experimental.pallas import tpu as pltpu
from jax.experimental.pallas import tpu_sc as plsc

N_BINS = 15
NC = 2
NS = 16
NW = NC * NS
L = 16

R_CHUNK = 1000
N_CHAINS = 4


def _sc_body(c, n, n_chunks, sm_ref, lab_ref, out_ref, buf, lbuf, acc, obuf):
    w = lax.axis_index("s") * NC + lax.axis_index("c")
    iota = lax.iota(jnp.int32, L)
    zeros = jnp.zeros((L,), jnp.float32)
    ones = jnp.full((L,), 1.0, jnp.float32)
    iota_c = iota * c
    iota_16 = iota * L

    for k in range(3 * L):
        acc[pl.ds(k * L, L)] = zeros

    chain = c // N_CHAINS
    n_full = R_CHUNK // L
    n_tail = R_CHUNK - n_full * L

    def do_group(base_row, row, valid_mask):
        addr0 = row * c
        curs = []
        curis = []
        for k in range(N_CHAINS):
            addr = addr0 + (k * chain)
            cur = jnp.full((L,), -1.0, jnp.float32)
            curi = addr
            for _ in range(chain):
                v = plsc.load_gather(buf, [addr])
                m = v > cur
                curi = jnp.where(m, addr, curi)
                cur = jnp.maximum(v, cur)
                addr = addr + 1
            curs.append(cur)
            curis.append(curi)
        cur, curi = curs[0], curis[0]
        for k in range(1, N_CHAINS):
            take = curs[k] > cur
            curi = jnp.where(take, curis[k], curi)
            cur = jnp.maximum(curs[k], cur)
        pred = curi - addr0
        conf = cur
        lab = lbuf[pl.ds(base_row, L)]
        accf = jnp.where(pred == lab, 1.0, 0.0).astype(jnp.float32)
        binv = jnp.minimum((conf * np.float32(N_BINS)).astype(jnp.int32),
                           N_BINS - 1)
        sidx = binv * L + iota
        plsc.addupdate_scatter(acc, [sidx], ones, mask=valid_mask)
        plsc.addupdate_scatter(acc, [sidx + 256], conf, mask=valid_mask)
        plsc.addupdate_scatter(acc, [sidx + 512], accf, mask=valid_mask)

    def chunk_body(j, _):
        t = w + NW * j
        sync = pltpu.sync_copy
        sync(sm_ref.at[pl.ds(t * (R_CHUNK * c), R_CHUNK * c)], buf)
        sync(lab_ref.at[pl.ds(t * R_CHUNK, R_CHUNK)],
             lbuf.at[pl.ds(0, R_CHUNK)])

        def group_body(g, _):
            base_row = g * L
            do_group(base_row, base_row + iota,
                     jnp.full((L,), True, jnp.bool_))
            return 0

        lax.fori_loop(0, n_full, group_body, 0)
        if n_tail:
            base_row = n_full * L
            row = jnp.minimum(base_row + iota, R_CHUNK - 1)
            do_group(base_row, row, iota < n_tail)
        return 0

    nj = (n_chunks // NW) + jnp.where(w < (n_chunks % NW), 1, 0)
    lax.fori_loop(0, nj, chunk_body, 0)

    for sec in range(3):
        tot = zeros
        for lane in range(L):
            tot = tot + plsc.load_gather(acc, [iota_16 + (sec * 256 + lane)])
        obuf[pl.ds(sec * L, L)] = tot
    pltpu.sync_copy(obuf, out_ref.at[w])


def _combine_body(n_total, p_ref, out_ref):
    x = p_ref[...]
    s = jnp.sum(x, axis=0)
    cnt = s[0:1]
    sconf = s[1:2]
    sacc = s[2:3]
    lane = lax.broadcasted_iota(jnp.int32, (1, L), 1)
    safe = jnp.maximum(cnt, 1.0)
    gap = jnp.abs(sconf / safe - sacc / safe) * (cnt / np.float32(n_total))
    gap = jnp.where((cnt > 0.0) & (lane < N_BINS), gap, 0.0)
    out_ref[...] = jnp.sum(gap).reshape(1, 1)


def kernel(softmaxes, labels):
    n, c = softmaxes.shape
    assert n % R_CHUNK == 0 and c % N_CHAINS == 0
    n_chunks = n // R_CHUNK

    mesh = plsc.VectorSubcoreMesh(core_axis_name="c", subcore_axis_name="s",
                                  num_cores=NC, num_subcores=NS)
    sc_fn = pl.kernel(
        functools.partial(_sc_body, c, n, n_chunks),
        out_type=jax.ShapeDtypeStruct((NW, 3 * L), jnp.float32),
        mesh=mesh,
        scratch_types=[
            pltpu.VMEM((R_CHUNK * c,), jnp.float32),
            pltpu.VMEM((R_CHUNK + L,), jnp.int32),
            pltpu.VMEM((3 * 256,), jnp.float32),
            pltpu.VMEM((3 * L,), jnp.float32),
        ],
        compiler_params=pltpu.CompilerParams(needs_layout_passes=False),
    )
    partials = sc_fn(softmaxes.reshape(-1), labels.astype(jnp.int32))

    out = pl.pallas_call(
        functools.partial(_combine_body, n),
        in_specs=[pl.BlockSpec((NW, 3, L), lambda: (0, 0, 0))],
        out_specs=pl.BlockSpec((1, 1), lambda: (0, 0)),
        out_shape=jax.ShapeDtypeStruct((1, 1), jnp.float32),
    )(partials.reshape(NW, 3, L))
    return out.reshape(1)


# speedup vs baseline: 1.4439x; 1.4439x over previous
import functools
import numpy as np
import jax
import jax.numpy as jnp
from jax import lax
from jax.experimental import pallas as pl
from jax.experimental.pallas import tpu as pltpu
from jax.experimental.pallas import tpu_sc as plsc

N_BINS = 15
NC = 2
NS = 16
NW = NC * NS
L = 16

R_CHUNK = 800
N_CHAINS = 4


def _sc_body(c, n, n_chunks, sm_ref, lab_ref, out_ref, buf, lbuf, acc, obuf):
    w = lax.axis_index("s") * NC + lax.axis_index("c")
    iota = lax.iota(jnp.int32, L)
    zeros = jnp.zeros((L,), jnp.float32)
    ones = jnp.full((L,), 1.0, jnp.float32)
    iota_16 = iota * L

    for k in range(3 * L):
        acc[pl.ds(k * L, L)] = zeros

    chain = c // N_CHAINS
    n_groups = R_CHUNK // L

    def do_group(base_row):
        row = base_row + iota
        curs = []
        curis = []
        for k in range(N_CHAINS):
            col = jnp.full((L,), k * chain, jnp.int32)
            cur = jnp.full((L,), -1.0, jnp.float32)
            curi = col
            for _ in range(chain):
                v = plsc.load_gather(buf, [row, col])
                m = v > cur
                curi = jnp.where(m, col, curi)
                cur = jnp.maximum(v, cur)
                col = col + 1
            curs.append(cur)
            curis.append(curi)
        cur, curi = curs[0], curis[0]
        for k in range(1, N_CHAINS):
            take = curs[k] > cur
            curi = jnp.where(take, curis[k], curi)
            cur = jnp.maximum(curs[k], cur)
        pred = curi
        conf = cur
        lab = lbuf[pl.ds(base_row, L)]
        accf = jnp.where(pred == lab, 1.0, 0.0).astype(jnp.float32)
        binv = jnp.minimum((conf * np.float32(N_BINS)).astype(jnp.int32),
                           N_BINS - 1)
        sidx = binv * L + iota
        plsc.addupdate_scatter(acc, [sidx], ones)
        plsc.addupdate_scatter(acc, [sidx + 256], conf)
        plsc.addupdate_scatter(acc, [sidx + 512], accf)

    def chunk_body(j, _):
        t = w + NW * j
        sync = pltpu.sync_copy
        sync(sm_ref.at[pl.ds(t * R_CHUNK, R_CHUNK)], buf)
        sync(lab_ref.at[pl.ds(t * R_CHUNK, R_CHUNK)],
             lbuf.at[pl.ds(0, R_CHUNK)])

        def group_body(g, _):
            do_group(g * L)
            return 0

        lax.fori_loop(0, n_groups, group_body, 0)
        return 0

    nj = (n_chunks // NW) + jnp.where(w < (n_chunks % NW), 1, 0)
    lax.fori_loop(0, nj, chunk_body, 0)

    for sec in range(3):
        tot = zeros
        for lane in range(L):
            tot = tot + plsc.load_gather(acc, [iota_16 + (sec * 256 + lane)])
        obuf[pl.ds(sec * L, L)] = tot
    pltpu.sync_copy(obuf, out_ref.at[w])


def _combine_body(n_total, p_ref, out_ref):
    x = p_ref[...]
    s = jnp.sum(x, axis=0)
    cnt = s[0:1]
    sconf = s[1:2]
    sacc = s[2:3]
    lane = lax.broadcasted_iota(jnp.int32, (1, L), 1)
    safe = jnp.maximum(cnt, 1.0)
    gap = jnp.abs(sconf / safe - sacc / safe) * (cnt / np.float32(n_total))
    gap = jnp.where((cnt > 0.0) & (lane < N_BINS), gap, 0.0)
    out_ref[...] = jnp.sum(gap).reshape(1, 1)


def kernel(softmaxes, labels):
    n, c = softmaxes.shape
    assert n % R_CHUNK == 0 and c % N_CHAINS == 0
    n_chunks = n // R_CHUNK

    mesh = plsc.VectorSubcoreMesh(core_axis_name="c", subcore_axis_name="s",
                                  num_cores=NC, num_subcores=NS)
    sc_fn = pl.kernel(
        functools.partial(_sc_body, c, n, n_chunks),
        out_type=jax.ShapeDtypeStruct((NW, 3 * L), jnp.float32),
        mesh=mesh,
        scratch_types=[
            pltpu.VMEM((R_CHUNK, c), jnp.float32),
            pltpu.VMEM((R_CHUNK + L,), jnp.int32),
            pltpu.VMEM((3 * 256,), jnp.float32),
            pltpu.VMEM((3 * L,), jnp.float32),
        ],
        compiler_params=pltpu.CompilerParams(needs_layout_passes=False,
                                             use_tc_tiling_on_sc=True),
    )
    partials = sc_fn(softmaxes, labels.astype(jnp.int32))

    out = pl.pallas_call(
        functools.partial(_combine_body, n),
        in_specs=[pl.BlockSpec((NW, 3, L), lambda: (0, 0, 0))],
        out_specs=pl.BlockSpec((1, 1), lambda: (0, 0)),
        out_shape=jax.ShapeDtypeStruct((1, 1), jnp.float32),
    )(partials.reshape(NW, 3, L))
    return out.reshape(1)


# speedup vs baseline: 3.8282x; 2.6513x over previous
import functools
import numpy as np
import jax
import jax.numpy as jnp
from jax import lax
from jax.experimental import pallas as pl
from jax.experimental.pallas import tpu as pltpu
from jax.experimental.pallas import tpu_sc as plsc

N_BINS = 15
NC = 2
NS = 16
NW = NC * NS
L = 16

R_CHUNK = 800
R_HALF = R_CHUNK // 2
N_CHAINS = 4


def _sc_body(c, n, n_chunks, sm_ref, lab_ref, out_ref,
             buf, lbuf, acc, obuf, sem_a, sem_b, sem_l):
    w = lax.axis_index("s") * NC + lax.axis_index("c")
    iota = lax.iota(jnp.int32, L)
    zeros = jnp.zeros((L,), jnp.float32)
    ones = jnp.full((L,), 1.0, jnp.float32)
    iota_16 = iota * L

    for k in range(3 * L):
        acc[pl.ds(k * L, L)] = zeros

    chain = c // N_CHAINS
    n_groups = R_CHUNK // L
    half_groups = R_HALF // L

    def copy_half(t, half, sem):
        src = sm_ref.at[pl.ds(t * R_CHUNK + half * R_HALF, R_HALF)]
        dst = buf.at[pl.ds(half * R_HALF, R_HALF)]
        return pltpu.make_async_copy(src, dst, sem)

    def copy_lab(t, sem):
        src = lab_ref.at[pl.ds(t * R_CHUNK, R_CHUNK)]
        dst = lbuf.at[pl.ds(0, R_CHUNK)]
        return pltpu.make_async_copy(src, dst, sem)

    def do_group(base_row):
        row = base_row + iota
        curs = []
        curis = []
        for k in range(N_CHAINS):
            cl = iota + (k * chain)
            cl = jnp.where(cl >= c, cl - c, cl)
            cur = jnp.full((L,), -1.0, jnp.float32)
            curi = cl
            for _ in range(chain):
                v = plsc.load_gather(buf, [row, cl])
                m = v > cur
                curi = jnp.where(m, cl, curi)
                cur = jnp.maximum(v, cur)
                cl = cl + 1
                cl = jnp.where(cl == c, 0, cl)
            curs.append(cur)
            curis.append(curi)
        cur, curi = curs[0], curis[0]
        for k in range(1, N_CHAINS):
            take = curs[k] > cur
            curi = jnp.where(take, curis[k], curi)
            cur = jnp.maximum(curs[k], cur)
        pred = curi
        conf = cur
        lab = lbuf[pl.ds(base_row, L)]
        accf = jnp.where(pred == lab, 1.0, 0.0).astype(jnp.float32)
        binv = jnp.minimum((conf * np.float32(N_BINS)).astype(jnp.int32),
                           N_BINS - 1)
        sidx = binv * L + iota
        plsc.addupdate_scatter(acc, [sidx], ones)
        plsc.addupdate_scatter(acc, [sidx + 256], conf)
        plsc.addupdate_scatter(acc, [sidx + 512], accf)

    t0 = w
    copy_half(t0, 0, sem_a).start()
    copy_lab(t0, sem_l).start()
    copy_half(t0, 1, sem_b).start()

    nj = (n_chunks // NW) + jnp.where(w < (n_chunks % NW), 1, 0)

    def chunk_body(j, _):
        t = w + NW * j
        t_next = t + NW
        copy_lab(t, sem_l).wait()
        copy_half(t, 0, sem_a).wait()

        def group_body(g, _):
            do_group(g * L)
            return 0

        lax.fori_loop(0, half_groups, group_body, 0)
        copy_half(t, 1, sem_b).wait()

        @pl.when(j + 1 < nj)
        def _():
            copy_half(t_next, 0, sem_a).start()

        def group_body2(g, _):
            do_group(g * L)
            return 0

        lax.fori_loop(half_groups, n_groups, group_body2, 0)

        @pl.when(j + 1 < nj)
        def _():
            copy_half(t_next, 1, sem_b).start()
            copy_lab(t_next, sem_l).start()
        return 0

    lax.fori_loop(0, nj, chunk_body, 0)

    for sec in range(3):
        tot = zeros
        for lane in range(L):
            tot = tot + plsc.load_gather(acc, [iota_16 + (sec * 256 + lane)])
        obuf[pl.ds(sec * L, L)] = tot
    pltpu.sync_copy(obuf, out_ref.at[w])


def _combine_body(n_total, p_ref, out_ref):
    x = p_ref[...]
    s = jnp.sum(x, axis=0)
    cnt = s[0:1]
    sconf = s[1:2]
    sacc = s[2:3]
    lane = lax.broadcasted_iota(jnp.int32, (1, L), 1)
    safe = jnp.maximum(cnt, 1.0)
    gap = jnp.abs(sconf / safe - sacc / safe) * (cnt / np.float32(n_total))
    gap = jnp.where((cnt > 0.0) & (lane < N_BINS), gap, 0.0)
    out_ref[...] = jnp.sum(gap).reshape(1, 1)


def kernel(softmaxes, labels):
    n, c = softmaxes.shape
    assert n % R_CHUNK == 0 and c % N_CHAINS == 0
    n_chunks = n // R_CHUNK

    mesh = plsc.VectorSubcoreMesh(core_axis_name="c", subcore_axis_name="s",
                                  num_cores=NC, num_subcores=NS)
    sc_fn = pl.kernel(
        functools.partial(_sc_body, c, n, n_chunks),
        out_type=jax.ShapeDtypeStruct((NW, 3 * L), jnp.float32),
        mesh=mesh,
        scratch_types=[
            pltpu.VMEM((R_CHUNK, c), jnp.float32),
            pltpu.VMEM((R_CHUNK + L,), jnp.int32),
            pltpu.VMEM((3 * 256,), jnp.float32),
            pltpu.VMEM((3 * L,), jnp.float32),
            pltpu.SemaphoreType.DMA,
            pltpu.SemaphoreType.DMA,
            pltpu.SemaphoreType.DMA,
        ],
        compiler_params=pltpu.CompilerParams(needs_layout_passes=False,
                                             use_tc_tiling_on_sc=True),
    )
    partials = sc_fn(softmaxes, labels.astype(jnp.int32))

    out = pl.pallas_call(
        functools.partial(_combine_body, n),
        in_specs=[pl.BlockSpec((NW, 3, L), lambda: (0, 0, 0))],
        out_specs=pl.BlockSpec((1, 1), lambda: (0, 0)),
        out_shape=jax.ShapeDtypeStruct((1, 1), jnp.float32),
    )(partials.reshape(NW, 3, L))
    return out.reshape(1)
